# baseline (speedup 1.0000x reference)
import functools

import jax
import jax.numpy as jnp
from jax import lax
from jax.experimental import pallas as pl
from jax.experimental.pallas import tpu as pltpu
from jax.experimental.pallas import tpu_sc as plsc

DIM = 32
N_EMBED = 8192
BETA = 0.25
TOK_BLK = 512
WIN = 4096
N_TOTAL = 16 * 32 * 32 * DIM
ST_BLK = 1024


def _argmin_body(z_ref, et2_ref, z2_ref, e2_ref, idx_ref):
    zb = z_ref[...]
    mm2 = jax.lax.dot_general(
        zb, et2_ref[...], (((1,), (0,)), ((), ())),
        preferred_element_type=jnp.float32)
    d = (z2_ref[...] + e2_ref[...]) - mm2

    big = jnp.int32(2 ** 30)
    acc_v = None
    acc_i = None
    for w in range(N_EMBED // WIN):
        dw = jax.lax.slice_in_dim(d, w * WIN, (w + 1) * WIN, axis=1)
        mv = jnp.min(dw, axis=1)
        ii = jax.lax.broadcasted_iota(jnp.int32, dw.shape, 1) + jnp.int32(w * WIN)
        mi = jnp.min(jnp.where(dw == mv[:, None], ii, big), axis=1)
        if w == 0:
            acc_v = mv.astype(jnp.bfloat16).astype(jnp.float32)
            acc_i = mi
        else:
            take = (mv < acc_v) | ((mv == acc_v) & (mi < acc_i))
            acc_i = jnp.where(take, mi, acc_i)
            acc_v = jnp.where(
                take, mv.astype(jnp.bfloat16).astype(jnp.float32), acc_v)
    idx_ref[0, 0, :] = acc_i


def _st_body(z_ref, q_ref, st_ref, loss_ref):
    i = pl.program_id(0)
    zb = z_ref[...]
    qb = q_ref[...]
    st_ref[...] = zb + (qb - zb)
    diff = qb - zb
    psum = jnp.sum(diff * diff).reshape(1, 1)

    @pl.when(i == 0)
    def _init():
        loss_ref[...] = jnp.zeros((1, 1), jnp.float32)

    loss_ref[...] += psum

    @pl.when(i == pl.num_programs(0) - 1)
    def _final():
        loss_ref[...] = jnp.float32(BETA / N_TOTAL + 1.0 / N_TOTAL) * loss_ref[...]


def _make_sc_gather(n_tok):
    info = plsc.get_sparse_core_info()
    nw = info.num_cores * info.num_subcores
    b_per_w = n_tok // nw
    chunk = 128
    n_chunk = b_per_w // chunk
    mesh = plsc.VectorSubcoreMesh(core_axis_name="c", subcore_axis_name="s")

    @functools.partial(
        pl.kernel, mesh=mesh,
        out_type=jax.ShapeDtypeStruct((n_tok, DIM), jnp.float32),
        compiler_params=pltpu.CompilerParams(use_tc_tiling_on_sc=False),
        scratch_types=[
            pltpu.VMEM((b_per_w,), jnp.int32),
            pltpu.VMEM((b_per_w, DIM), jnp.float32),
            pltpu.SemaphoreType.DMA,
        ],
    )
    def gather(table_hbm, idx_hbm, out_hbm, idx_v, rows_v, sem):
        wid = lax.axis_index("s") * info.num_cores + lax.axis_index("c")
        base = wid * b_per_w
        pltpu.sync_copy(idx_hbm.at[pl.ds(base, b_per_w)], idx_v)
        for c in range(n_chunk):
            pltpu.async_copy(
                table_hbm.at[idx_v.at[pl.ds(c * chunk, chunk)]],
                rows_v.at[pl.ds(c * chunk, chunk)], sem).wait()
        pltpu.sync_copy(rows_v, out_hbm.at[pl.ds(base, b_per_w)])

    return gather


def kernel(z, embedding_weight):
    B, C, H, W = z.shape
    zp = jnp.transpose(z, (0, 2, 3, 1))
    z_flat = zp.reshape(-1, DIM)
    n_tok = z_flat.shape[0]
    n_blk = n_tok // TOK_BLK
    et2 = (2.0 * embedding_weight).T
    z2 = jnp.sum(z_flat ** 2, axis=1, keepdims=True)
    e2 = jnp.sum(embedding_weight ** 2, axis=1).reshape(1, N_EMBED)
    table16 = embedding_weight.astype(jnp.bfloat16).astype(jnp.float32)

    idx3 = pl.pallas_call(
        _argmin_body,
        grid=(n_blk,),
        in_specs=[
            pl.BlockSpec((TOK_BLK, DIM), lambda i: (i, 0)),
            pl.BlockSpec((DIM, N_EMBED), lambda i: (0, 0)),
            pl.BlockSpec((TOK_BLK, 1), lambda i: (i, 0)),
            pl.BlockSpec((1, N_EMBED), lambda i: (0, 0)),
        ],
        out_specs=pl.BlockSpec((1, 1, TOK_BLK), lambda i: (i, 0, 0)),
        out_shape=jax.ShapeDtypeStruct((n_blk, 1, TOK_BLK), jnp.int32),
    )(z_flat, et2, z2, e2)

    indices_flat = idx3.reshape(n_tok)
    zq = _make_sc_gather(n_tok)(table16, indices_flat)

    zq_st, loss = pl.pallas_call(
        _st_body,
        grid=(n_tok // ST_BLK,),
        in_specs=[
            pl.BlockSpec((ST_BLK, DIM), lambda i: (i, 0)),
            pl.BlockSpec((ST_BLK, DIM), lambda i: (i, 0)),
        ],
        out_specs=[
            pl.BlockSpec((ST_BLK, DIM), lambda i: (i, 0)),
            pl.BlockSpec((1, 1), lambda i: (0, 0)),
        ],
        out_shape=[
            jax.ShapeDtypeStruct((n_tok, DIM), jnp.float32),
            jax.ShapeDtypeStruct((1, 1), jnp.float32),
        ],
    )(z_flat, zq)

    indices = indices_flat.reshape(B, H, W)
    z_q_out = jnp.transpose(zq_st.reshape(B, H, W, C), (0, 3, 1, 2))
    return (z_q_out, loss[0, 0], indices)


# speedup vs baseline: 1.0918x; 1.0918x over previous
import functools

import jax
import jax.numpy as jnp
from jax import lax
from jax.experimental import pallas as pl
from jax.experimental.pallas import tpu as pltpu
from jax.experimental.pallas import tpu_sc as plsc

DIM = 32
N_EMBED = 8192
BETA = 0.25
TOK_BLK = 512
WIN = 4096
N_TOTAL = 16 * 32 * 32 * DIM
ST_BLK = 1024


def _argmin_body(z_ref, et_ref, z2_ref, e2_ref, idx_ref):
    zb = z_ref[...]
    mm = jax.lax.dot_general(
        zb, et_ref[...], (((1,), (0,)), ((), ())),
        preferred_element_type=jnp.float32)
    d = (z2_ref[...] + e2_ref[...]) - 2.0 * mm

    big = jnp.int32(2 ** 30)
    acc_v = None
    acc_i = None
    for w in range(N_EMBED // WIN):
        dw = jax.lax.slice_in_dim(d, w * WIN, (w + 1) * WIN, axis=1)
        mv = jnp.min(dw, axis=1)
        mi = jnp.argmin(dw, axis=1).astype(jnp.int32) + jnp.int32(w * WIN)
        if w == 0:
            acc_v = mv.astype(jnp.bfloat16).astype(jnp.float32)
            acc_i = mi
        else:
            take = (mv < acc_v) | ((mv == acc_v) & (mi < acc_i))
            acc_i = jnp.where(take, mi, acc_i)
            acc_v = jnp.where(
                take, mv.astype(jnp.bfloat16).astype(jnp.float32), acc_v)
    idx_ref[0, 0, :] = acc_i


def _st_body(z_ref, q_ref, st_ref, loss_ref):
    i = pl.program_id(0)
    zb = z_ref[...]
    qb = q_ref[...]
    st_ref[...] = zb + (qb - zb)
    diff = qb - zb
    psum = jnp.sum(diff * diff).reshape(1, 1)

    @pl.when(i == 0)
    def _init():
        loss_ref[...] = jnp.zeros((1, 1), jnp.float32)

    loss_ref[...] += psum

    @pl.when(i == pl.num_programs(0) - 1)
    def _final():
        loss_ref[...] = jnp.float32(BETA / N_TOTAL + 1.0 / N_TOTAL) * loss_ref[...]


def _make_sc_gather(n_tok):
    info = plsc.get_sparse_core_info()
    nw = info.num_cores * info.num_subcores
    b_per_w = n_tok // nw
    chunk = 128
    n_chunk = b_per_w // chunk
    mesh = plsc.VectorSubcoreMesh(core_axis_name="c", subcore_axis_name="s")

    @functools.partial(
        pl.kernel, mesh=mesh,
        out_type=jax.ShapeDtypeStruct((n_tok, DIM), jnp.float32),
        compiler_params=pltpu.CompilerParams(use_tc_tiling_on_sc=False),
        scratch_types=[
            pltpu.VMEM((b_per_w,), jnp.int32),
            pltpu.VMEM((b_per_w, DIM), jnp.float32),
            pltpu.SemaphoreType.DMA,
        ],
    )
    def gather(table_hbm, idx_hbm, out_hbm, idx_v, rows_v, sem):
        wid = lax.axis_index("s") * info.num_cores + lax.axis_index("c")
        base = wid * b_per_w
        pltpu.sync_copy(idx_hbm.at[pl.ds(base, b_per_w)], idx_v)
        for c in range(n_chunk):
            pltpu.async_copy(
                table_hbm.at[idx_v.at[pl.ds(c * chunk, chunk)]],
                rows_v.at[pl.ds(c * chunk, chunk)], sem).wait()
        pltpu.sync_copy(rows_v, out_hbm.at[pl.ds(base, b_per_w)])

    return gather


def kernel(z, embedding_weight):
    B, C, H, W = z.shape
    zp = jnp.transpose(z, (0, 2, 3, 1))
    z_flat = zp.reshape(-1, DIM)
    n_tok = z_flat.shape[0]
    n_blk = n_tok // TOK_BLK
    et = embedding_weight.T
    z2 = jnp.sum(z_flat ** 2, axis=1, keepdims=True)
    e2 = jnp.sum(embedding_weight ** 2, axis=1).reshape(1, N_EMBED)
    table16 = embedding_weight.astype(jnp.bfloat16).astype(jnp.float32)

    idx3 = pl.pallas_call(
        _argmin_body,
        grid=(n_blk,),
        in_specs=[
            pl.BlockSpec((TOK_BLK, DIM), lambda i: (i, 0)),
            pl.BlockSpec((DIM, N_EMBED), lambda i: (0, 0)),
            pl.BlockSpec((TOK_BLK, 1), lambda i: (i, 0)),
            pl.BlockSpec((1, N_EMBED), lambda i: (0, 0)),
        ],
        out_specs=pl.BlockSpec((1, 1, TOK_BLK), lambda i: (i, 0, 0)),
        out_shape=jax.ShapeDtypeStruct((n_blk, 1, TOK_BLK), jnp.int32),
    )(z_flat, et, z2, e2)

    indices_flat = idx3.reshape(n_tok)
    zq = _make_sc_gather(n_tok)(table16, indices_flat)

    zq_st, loss = pl.pallas_call(
        _st_body,
        grid=(n_tok // ST_BLK,),
        in_specs=[
            pl.BlockSpec((ST_BLK, DIM), lambda i: (i, 0)),
            pl.BlockSpec((ST_BLK, DIM), lambda i: (i, 0)),
        ],
        out_specs=[
            pl.BlockSpec((ST_BLK, DIM), lambda i: (i, 0)),
            pl.BlockSpec((1, 1), lambda i: (0, 0)),
        ],
        out_shape=[
            jax.ShapeDtypeStruct((n_tok, DIM), jnp.float32),
            jax.ShapeDtypeStruct((1, 1), jnp.float32),
        ],
    )(z_flat, zq)

    indices = indices_flat.reshape(B, H, W)
    z_q_out = jnp.transpose(zq_st.reshape(B, H, W, C), (0, 3, 1, 2))
    return (z_q_out, loss[0, 0], indices)
